# baseline (speedup 1.0000x reference)
import functools

import jax
import jax.numpy as jnp
from jax import lax
from jax.experimental import pallas as pl
from jax.experimental.pallas import tpu as pltpu
from jax.experimental.pallas import tpu_sc as plsc

B = 4096
DIM = 32
HIST = 50
NHIST = B * HIST
VOCAB = 1000000

NC = 2
NS = 16
NW = NC * NS

CH = 128
BPW_S = B // NW
BPW_H = NHIST // NW
NCH_H = BPW_H // CH

NSLAB = (VOCAB + 127) // 128
NSLAB_FULL = VOCAB // 128
TAIL = VOCAB - NSLAB_FULL * 128
NSLAB_W = (NSLAB + NW - 1) // NW


def _sc_prep_body(emt, xu0, xi1, eut, ect,
                  emf, gu, gic,
                  idxv, slab, tslab, wbuf, obuf):
    wid = lax.axis_index("s") * NC + lax.axis_index("c")
    sbase = wid * BPW_S
    iota16 = lax.iota(jnp.int32, 16)

    def slab_gather(x_hbm, et, out_hbm):
        pltpu.sync_copy(x_hbm.at[pl.ds(sbase, BPW_S)], idxv)

        def group(g, carry):
            chunk = idxv[pl.ds(g * 16, 16)]
            for lane in range(16):
                idx = jnp.max(jnp.where(iota16 == lane, chunk, 0))
                i0 = (idx // 128) * 128
                lv = jnp.full((16,), idx - i0, jnp.int32)
                pltpu.sync_copy(et.at[:, pl.ds(i0, 128)], slab)
                row = g * 16 + lane
                for k in range(2):
                    v = plsc.load_gather(slab, [iota16 + 16 * k, lv])
                    obuf[row, pl.ds(16 * k, 16)] = v
            return carry

        lax.fori_loop(0, BPW_S // 16, group, 0)
        pltpu.sync_copy(obuf, out_hbm.at[pl.ds(sbase, BPW_S)])

    slab_gather(xu0, eut, gu)
    slab_gather(xi1, ect, gic)

    def slab_step(i, carry):
        s = wid + i * NW

        @pl.when(s < NSLAB)
        def _():
            pltpu.sync_copy(emt.at[:, pl.ds(s * 128, 128)], tslab)

            def row(j, c):
                jv = jnp.full((16,), j, jnp.int32)
                for k in range(2):
                    v = plsc.load_gather(tslab, [iota16 + 16 * k, jv])
                    wbuf[pl.ds(j * DIM + 16 * k, 16)] = v
                return c

            lax.fori_loop(0, 128, row, 0)
            base = s * 128 * DIM

            @pl.when(s < NSLAB_FULL)
            def _full():
                pltpu.sync_copy(wbuf, emf.at[pl.ds(base, 128 * DIM)])

            @pl.when(s == NSLAB_FULL)
            def _tail():
                pltpu.sync_copy(wbuf.at[pl.ds(0, TAIL * DIM)],
                                emf.at[pl.ds(base, TAIL * DIM)])

        return carry

    lax.fori_loop(0, NSLAB_W, slab_step, 0)


_sc_prep = functools.partial(
    pl.kernel,
    out_type=[
        jax.ShapeDtypeStruct((VOCAB * DIM,), jnp.float32),
        jax.ShapeDtypeStruct((B, DIM), jnp.float32),
        jax.ShapeDtypeStruct((B, DIM), jnp.float32),
    ],
    mesh=plsc.VectorSubcoreMesh(core_axis_name="c", subcore_axis_name="s"),
    scratch_types=[
        pltpu.VMEM((BPW_S,), jnp.int32),
        pltpu.VMEM((DIM, 128), jnp.float32),
        pltpu.VMEM((DIM, 128), jnp.float32),
        pltpu.VMEM((128 * DIM,), jnp.float32),
        pltpu.VMEM((BPW_S, DIM), jnp.float32),
    ],
    compiler_params=pltpu.CompilerParams(use_tc_tiling_on_sc=True,
                                         needs_layout_passes=False),
)(_sc_prep_body)


def _sc_movie_body(xu1f, xi0, em, gh, gim, idxb, rowb, sem):
    wid = lax.axis_index("s") * NC + lax.axis_index("c")

    def row_chunk(idx_hbm, out_hbm, base):
        pltpu.sync_copy(idx_hbm.at[pl.ds(base, CH)], idxb)
        pltpu.async_copy(em.at[idxb], rowb, sem).wait()
        pltpu.sync_copy(rowb, out_hbm.at[pl.ds(base, CH)])

    row_chunk(xi0, gim, wid * BPW_S)

    hbase = wid * BPW_H

    def step(i, carry):
        row_chunk(xu1f, gh, hbase + i * CH)
        return carry

    lax.fori_loop(0, NCH_H, step, 0)


_sc_movie = functools.partial(
    pl.kernel,
    out_type=[
        jax.ShapeDtypeStruct((NHIST, DIM), jnp.float32),
        jax.ShapeDtypeStruct((B, DIM), jnp.float32),
    ],
    mesh=plsc.VectorSubcoreMesh(core_axis_name="c", subcore_axis_name="s"),
    scratch_types=[
        pltpu.VMEM((CH,), jnp.int32),
        pltpu.VMEM((CH, DIM), jnp.float32),
        pltpu.SemaphoreType.DMA,
    ],
    compiler_params=pltpu.CompilerParams(use_tc_tiling_on_sc=False),
)(_sc_movie_body)


BLK = 512


def _tc_body(gu, gh, gim, gic,
             wu1, bu1, wu2, bu2, wi1, bi1, wi2, bi2,
             out):
    f32 = jnp.float32
    uh = (
        jnp.dot(gu[...], wu1[0:DIM, :], preferred_element_type=f32)
        + jnp.dot(gh[...], wu1[DIM:, :], preferred_element_type=f32)
        + bu1[...]
    )
    uh = jnp.maximum(uh, 0.0)
    uo = jnp.dot(uh, wu2[...], preferred_element_type=f32) + bu2[...]

    ih = (
        jnp.dot(gim[...], wi1[0:DIM, :], preferred_element_type=f32)
        + jnp.dot(gic[...], wi1[DIM:, :], preferred_element_type=f32)
        + bi1[...]
    )
    ih = jnp.maximum(ih, 0.0)
    io = jnp.dot(ih, wi2[...], preferred_element_type=f32) + bi2[...]

    s = jnp.sum(uo * io, axis=1, keepdims=True)
    out[...] = 1.0 / (1.0 + jnp.exp(-s))


def _tc_towers(gu, gh, gim, gic, Wu1, bu1, Wu2, bu2, Wi1, bi1, Wi2, bi2):
    full = lambda shape: pl.BlockSpec(shape, lambda i: (0, 0))
    return pl.pallas_call(
        _tc_body,
        grid=(B // BLK,),
        in_specs=[
            pl.BlockSpec((BLK, DIM), lambda i: (i, 0)),
            pl.BlockSpec((BLK, HIST * DIM), lambda i: (i, 0)),
            pl.BlockSpec((BLK, DIM), lambda i: (i, 0)),
            pl.BlockSpec((BLK, DIM), lambda i: (i, 0)),
            full(Wu1.shape), full((1, 64)), full(Wu2.shape), full((1, 32)),
            full(Wi1.shape), full((1, 64)), full(Wi2.shape), full((1, 32)),
        ],
        out_specs=pl.BlockSpec((BLK, 1), lambda i: (i, 0)),
        out_shape=jax.ShapeDtypeStruct((B, 1), jnp.float32),
    )(gu, gh, gim, gic,
      Wu1, bu1.reshape(1, 64), Wu2, bu2.reshape(1, 32),
      Wi1, bi1.reshape(1, 64), Wi2, bi2.reshape(1, 32))


@jax.jit
def kernel(X_user_0, X_user_1, X_item_0, X_item_1, E_user, E_movie, E_cate,
           Wu1, bu1, Wu2, bu2, Wi1, bi1, Wi2, bi2):
    xu1f = X_user_1.reshape(NHIST)
    emf, gu, gic = _sc_prep(E_movie.T, X_user_0, X_item_1,
                            E_user.T, E_cate.T)
    gh, gim = _sc_movie(xu1f, X_item_0, emf.reshape(VOCAB, DIM))
    gh = gh.reshape(B, HIST * DIM)
    out = _tc_towers(gu, gh, gim, gic,
                     Wu1, bu1, Wu2, bu2, Wi1, bi1, Wi2, bi2)
    return out.reshape(B)


# speedup vs baseline: 1.6535x; 1.6535x over previous
import functools

import jax
import jax.numpy as jnp
from jax import lax
from jax.experimental import pallas as pl
from jax.experimental.pallas import tpu as pltpu
from jax.experimental.pallas import tpu_sc as plsc

B = 4096
DIM = 32
HIST = 50
NHIST = B * HIST

NC = 2
NS = 16
NW = NC * NS

CH = 128
BPW_S = B // NW
BPW_H = NHIST // NW
NCH_H = BPW_H // CH


def _sc_small_body(xu0, xi1, eut, ect, gu, gic, idxv, slab, obuf):
    wid = lax.axis_index("s") * NC + lax.axis_index("c")
    sbase = wid * BPW_S
    iota16 = lax.iota(jnp.int32, 16)

    def slab_gather(x_hbm, et, out_hbm):
        pltpu.sync_copy(x_hbm.at[pl.ds(sbase, BPW_S)], idxv)

        def group(g, carry):
            chunk = idxv[pl.ds(g * 16, 16)]
            for lane in range(16):
                idx = jnp.max(jnp.where(iota16 == lane, chunk, 0))
                i0 = (idx // 128) * 128
                lv = jnp.full((16,), idx - i0, jnp.int32)
                pltpu.sync_copy(et.at[:, pl.ds(i0, 128)], slab)
                row = g * 16 + lane
                for k in range(2):
                    v = plsc.load_gather(slab, [iota16 + 16 * k, lv])
                    obuf[row, pl.ds(16 * k, 16)] = v
            return carry

        lax.fori_loop(0, BPW_S // 16, group, 0)
        pltpu.sync_copy(obuf, out_hbm.at[pl.ds(sbase, BPW_S)])

    slab_gather(xu0, eut, gu)
    slab_gather(xi1, ect, gic)


_sc_small = functools.partial(
    pl.kernel,
    out_type=[
        jax.ShapeDtypeStruct((B, DIM), jnp.float32),
        jax.ShapeDtypeStruct((B, DIM), jnp.float32),
    ],
    mesh=plsc.VectorSubcoreMesh(core_axis_name="c", subcore_axis_name="s"),
    scratch_types=[
        pltpu.VMEM((BPW_S,), jnp.int32),
        pltpu.VMEM((DIM, 128), jnp.float32),
        pltpu.VMEM((BPW_S, DIM), jnp.float32),
    ],
    compiler_params=pltpu.CompilerParams(use_tc_tiling_on_sc=True,
                                         needs_layout_passes=False),
)(_sc_small_body)


def _sc_movie_body(xu1f, xi0, em, gh, gim, idxb, rowb, sem):
    wid = lax.axis_index("s") * NC + lax.axis_index("c")

    def row_chunk(idx_hbm, out_hbm, base):
        pltpu.sync_copy(idx_hbm.at[pl.ds(base, CH)], idxb)
        pltpu.async_copy(em.at[idxb], rowb, sem).wait()
        pltpu.sync_copy(rowb, out_hbm.at[pl.ds(base, CH)])

    row_chunk(xi0, gim, wid * BPW_S)

    hbase = wid * BPW_H

    def step(i, carry):
        row_chunk(xu1f, gh, hbase + i * CH)
        return carry

    lax.fori_loop(0, NCH_H, step, 0)


_sc_movie = functools.partial(
    pl.kernel,
    out_type=[
        jax.ShapeDtypeStruct((NHIST, DIM), jnp.float32),
        jax.ShapeDtypeStruct((B, DIM), jnp.float32),
    ],
    mesh=plsc.VectorSubcoreMesh(core_axis_name="c", subcore_axis_name="s"),
    scratch_types=[
        pltpu.VMEM((CH,), jnp.int32),
        pltpu.VMEM((CH, DIM), jnp.float32),
        pltpu.SemaphoreType.DMA,
    ],
    compiler_params=pltpu.CompilerParams(use_tc_tiling_on_sc=False),
)(_sc_movie_body)


BLK = 512


def _tc_body(gu, gh, gim, gic,
             wu1, bu1, wu2, bu2, wi1, bi1, wi2, bi2,
             out):
    f32 = jnp.float32
    uh = (
        jnp.dot(gu[...], wu1[0:DIM, :], preferred_element_type=f32)
        + jnp.dot(gh[...], wu1[DIM:, :], preferred_element_type=f32)
        + bu1[...]
    )
    uh = jnp.maximum(uh, 0.0)
    uo = jnp.dot(uh, wu2[...], preferred_element_type=f32) + bu2[...]

    ih = (
        jnp.dot(gim[...], wi1[0:DIM, :], preferred_element_type=f32)
        + jnp.dot(gic[...], wi1[DIM:, :], preferred_element_type=f32)
        + bi1[...]
    )
    ih = jnp.maximum(ih, 0.0)
    io = jnp.dot(ih, wi2[...], preferred_element_type=f32) + bi2[...]

    s = jnp.sum(uo * io, axis=1, keepdims=True)
    out[...] = 1.0 / (1.0 + jnp.exp(-s))


def _tc_towers(gu, gh, gim, gic, Wu1, bu1, Wu2, bu2, Wi1, bi1, Wi2, bi2):
    full = lambda shape: pl.BlockSpec(shape, lambda i: (0, 0))
    return pl.pallas_call(
        _tc_body,
        grid=(B // BLK,),
        in_specs=[
            pl.BlockSpec((BLK, DIM), lambda i: (i, 0)),
            pl.BlockSpec((BLK, HIST * DIM), lambda i: (i, 0)),
            pl.BlockSpec((BLK, DIM), lambda i: (i, 0)),
            pl.BlockSpec((BLK, DIM), lambda i: (i, 0)),
            full(Wu1.shape), full((1, 64)), full(Wu2.shape), full((1, 32)),
            full(Wi1.shape), full((1, 64)), full(Wi2.shape), full((1, 32)),
        ],
        out_specs=pl.BlockSpec((BLK, 1), lambda i: (i, 0)),
        out_shape=jax.ShapeDtypeStruct((B, 1), jnp.float32),
    )(gu, gh, gim, gic,
      Wu1, bu1.reshape(1, 64), Wu2, bu2.reshape(1, 32),
      Wi1, bi1.reshape(1, 64), Wi2, bi2.reshape(1, 32))


@jax.jit
def kernel(X_user_0, X_user_1, X_item_0, X_item_1, E_user, E_movie, E_cate,
           Wu1, bu1, Wu2, bu2, Wi1, bi1, Wi2, bi2):
    xu1f = X_user_1.reshape(NHIST)
    gu, gic = _sc_small(X_user_0, X_item_1, E_user.T, E_cate.T)
    gh, gim = _sc_movie(xu1f, X_item_0, E_movie)
    gh = gh.reshape(B, HIST * DIM)
    out = _tc_towers(gu, gh, gim, gic,
                     Wu1, bu1, Wu2, bu2, Wi1, bi1, Wi2, bi2)
    return out.reshape(B)


# speedup vs baseline: 2.0082x; 1.2146x over previous
import functools

import jax
import jax.numpy as jnp
from jax import lax
from jax.experimental import pallas as pl
from jax.experimental.pallas import tpu as pltpu
from jax.experimental.pallas import tpu_sc as plsc

B = 4096
DIM = 32
HIST = 50
NHIST = B * HIST

NC = 2
NS = 16
NW = NC * NS

CH = 128
BPW_S = B // NW
BPW_H = NHIST // NW
NCH_H = BPW_H // CH


def _sc_small_body(xu0, xi1, eut, ect, gu, gic, idxv, slab, obuf):
    wid = lax.axis_index("s") * NC + lax.axis_index("c")
    sbase = wid * BPW_S
    iota16 = lax.iota(jnp.int32, 16)

    def slab_gather(x_hbm, et, out_hbm):
        pltpu.sync_copy(x_hbm.at[pl.ds(sbase, BPW_S)], idxv)

        def group(g, carry):
            chunk = idxv[pl.ds(g * 16, 16)]
            for lane in range(16):
                idx = jnp.max(jnp.where(iota16 == lane, chunk, 0))
                i0 = (idx // 128) * 128
                lv = jnp.full((16,), idx - i0, jnp.int32)
                pltpu.sync_copy(et.at[:, pl.ds(i0, 128)], slab)
                row = g * 16 + lane
                for k in range(2):
                    v = plsc.load_gather(slab, [iota16 + 16 * k, lv])
                    obuf[row, pl.ds(16 * k, 16)] = v
            return carry

        lax.fori_loop(0, BPW_S // 16, group, 0)
        pltpu.sync_copy(obuf, out_hbm.at[pl.ds(sbase, BPW_S)])

    slab_gather(xu0, eut, gu)
    slab_gather(xi1, ect, gic)


_sc_small = functools.partial(
    pl.kernel,
    out_type=[
        jax.ShapeDtypeStruct((B, DIM), jnp.float32),
        jax.ShapeDtypeStruct((B, DIM), jnp.float32),
    ],
    mesh=plsc.VectorSubcoreMesh(core_axis_name="c", subcore_axis_name="s"),
    scratch_types=[
        pltpu.VMEM((BPW_S,), jnp.int32),
        pltpu.VMEM((DIM, 128), jnp.float32),
        pltpu.VMEM((BPW_S, DIM), jnp.float32),
    ],
    compiler_params=pltpu.CompilerParams(use_tc_tiling_on_sc=True,
                                         needs_layout_passes=False),
)(_sc_small_body)


def _sc_movie_body(xu1f, xi0, em, gh, gim, idxb, rowb, sem):
    wid = lax.axis_index("s") * NC + lax.axis_index("c")

    def row_chunk(idx_hbm, out_hbm, base):
        pltpu.sync_copy(idx_hbm.at[pl.ds(base, CH)], idxb)
        pltpu.async_copy(em.at[idxb], rowb, sem).wait()
        pltpu.sync_copy(rowb, out_hbm.at[pl.ds(base, CH)])

    row_chunk(xi0, gim, wid * BPW_S)

    hbase = wid * BPW_H

    def step(i, carry):
        row_chunk(xu1f, gh, hbase + i * CH)
        return carry

    lax.fori_loop(0, NCH_H, step, 0)


_sc_movie = functools.partial(
    pl.kernel,
    out_type=[
        jax.ShapeDtypeStruct((NHIST, DIM), jnp.float32),
        jax.ShapeDtypeStruct((B, DIM), jnp.float32),
    ],
    mesh=plsc.VectorSubcoreMesh(core_axis_name="c", subcore_axis_name="s"),
    scratch_types=[
        pltpu.VMEM((CH,), jnp.int32),
        pltpu.VMEM((CH, DIM), jnp.float32),
        pltpu.SemaphoreType.DMA,
    ],
    compiler_params=pltpu.CompilerParams(use_tc_tiling_on_sc=False),
)(_sc_movie_body)


BLK = 512


def _tc_body(gu, gh, gim, gic,
             wu1, bu1, wu2, bu2, wi1, bi1, wi2, bi2,
             out):
    f32 = jnp.float32
    uh = (
        jnp.dot(gu[...], wu1[0:DIM, :], preferred_element_type=f32)
        + jnp.dot(gh[...], wu1[DIM:, :], preferred_element_type=f32)
        + bu1[...]
    )
    uh = jnp.maximum(uh, 0.0)
    uo = jnp.dot(uh, wu2[...], preferred_element_type=f32) + bu2[...]

    ih = (
        jnp.dot(gim[...], wi1[0:DIM, :], preferred_element_type=f32)
        + jnp.dot(gic[...], wi1[DIM:, :], preferred_element_type=f32)
        + bi1[...]
    )
    ih = jnp.maximum(ih, 0.0)
    io = jnp.dot(ih, wi2[...], preferred_element_type=f32) + bi2[...]

    s = jnp.sum(uo * io, axis=1, keepdims=True)
    out[...] = 1.0 / (1.0 + jnp.exp(-s))


def _tc_towers(gu, gh, gim, gic, Wu1, bu1, Wu2, bu2, Wi1, bi1, Wi2, bi2):
    full = lambda shape: pl.BlockSpec(shape, lambda i: (0, 0))
    return pl.pallas_call(
        _tc_body,
        grid=(B // BLK,),
        in_specs=[
            pl.BlockSpec((BLK, DIM), lambda i: (i, 0)),
            pl.BlockSpec((BLK, HIST * DIM), lambda i: (i, 0)),
            pl.BlockSpec((BLK, DIM), lambda i: (i, 0)),
            pl.BlockSpec((BLK, DIM), lambda i: (i, 0)),
            full(Wu1.shape), full((1, 64)), full(Wu2.shape), full((1, 32)),
            full(Wi1.shape), full((1, 64)), full(Wi2.shape), full((1, 32)),
        ],
        out_specs=pl.BlockSpec((BLK, 1), lambda i: (i, 0)),
        out_shape=jax.ShapeDtypeStruct((B, 1), jnp.float32),
    )(gu, gh, gim, gic,
      Wu1, bu1.reshape(1, 64), Wu2, bu2.reshape(1, 32),
      Wi1, bi1.reshape(1, 64), Wi2, bi2.reshape(1, 32))


@jax.jit
def kernel(X_user_0, X_user_1, X_item_0, X_item_1, E_user, E_movie, E_cate,
           Wu1, bu1, Wu2, bu2, Wi1, bi1, Wi2, bi2):
    xu1f = X_user_1.reshape(NHIST)
    gu, gic = _sc_small(X_user_0, X_item_1, E_user.T, E_cate.T)
    dep = (gu[0, 0] * 0.0).astype(jnp.int32)
    gh, gim = _sc_movie(xu1f, X_item_0 + dep, E_movie)
    gh = gh.reshape(B, HIST * DIM)
    out = _tc_towers(gu, gh, gim, gic,
                     Wu1, bu1, Wu2, bu2, Wi1, bi1, Wi2, bi2)
    return out.reshape(B)
